# baseline (speedup 1.0000x reference)
import functools

import jax
import jax.numpy as jnp
from jax import lax
from jax.experimental import pallas as pl
from jax.experimental.pallas import tpu as pltpu
from jax.experimental.pallas import tpu_sc as plsc

DIM = 64
LANES = 16
BLK = 128
SUB = 8


def _make_kernel(B, L, vocab):
    info = plsc.get_sparse_core_info()
    num_workers = info.num_cores * info.num_subcores
    nB = B // BLK
    assert nB * BLK == B
    n_blocks = L * nB
    per_w = n_blocks // num_workers
    assert per_w * num_workers == n_blocks
    assert per_w % 2 == 0

    mesh = plsc.VectorSubcoreMesh(core_axis_name="c", subcore_axis_name="s")

    @functools.partial(
        pl.kernel,
        mesh=mesh,
        out_type=jax.ShapeDtypeStruct((L * DIM // SUB, nB, SUB, BLK),
                                      jnp.float32),
        scratch_types=[
            pltpu.VMEM((per_w, BLK), jnp.int32),
            pltpu.VMEM((4, BLK, DIM), jnp.float32),
            pltpu.VMEM((SUB, SUB, BLK), jnp.float32),
            pltpu.SemaphoreType.DMA((2,)),
        ],
        compiler_params=pltpu.CompilerParams(
            use_tc_tiling_on_sc=False, needs_layout_passes=False),
    )
    def k(idx_hbm, table_hbm, out_hbm, idx_v, rows, out_v, semg):
        wid = lax.axis_index("s") * info.num_cores + lax.axis_index("c")
        lane = lax.iota(jnp.int32, LANES)

        HB = per_w // 2

        def load_idx_half(h):
            pltpu.sync_copy(
                idx_hbm.at[pl.ds(
                    pl.multiple_of(wid * 2 * per_w + h * 2 * HB, 8), 2 * HB)],
                idx_v)

        load_idx_half(0)

        def fire(kb, par):
            r = lax.rem(2 * kb, 2 * HB)
            pltpu.async_copy(table_hbm.at[idx_v.at[r]],
                             rows.at[2 * par], semg.at[par])
            pltpu.async_copy(table_hbm.at[idx_v.at[r + 1]],
                             rows.at[2 * par + 1], semg.at[par])

        def wait(kb, par):
            r = lax.rem(2 * kb, 2 * HB)
            pltpu.make_async_copy(table_hbm.at[idx_v.at[r]],
                                  rows.at[2 * par], semg.at[par]).wait()
            pltpu.make_async_copy(table_hbm.at[idx_v.at[r + 1]],
                                  rows.at[2 * par + 1], semg.at[par]).wait()

        def out_slice(kb):
            gb = wid * per_w + kb
            l = gb // nB
            c = gb % nB
            return out_hbm.at[pl.ds(pl.multiple_of(l * SUB, 8), SUB), c]

        fire(0, 0)

        def body(kb, carry):
            par = lax.rem(kb, 2)
            nxt = 1 - par

            @pl.when(jnp.logical_and(kb < per_w - 1, kb != HB - 1))
            def _():
                fire(kb + 1, nxt)

            wait(kb, par)

            @pl.when(kb == HB - 1)
            def _():
                load_idx_half(1)
                fire(kb + 1, nxt)

            r0 = rows.at[2 * par]
            r1 = rows.at[2 * par + 1]
            tvs = [lane + t0 for t0 in range(0, BLK, LANES)]

            def diag_body(j, c2):
                dv = ((lane + j) & (LANES - 1)) + (j & (DIM - LANES))
                dhi = lax.shift_right_logical(dv, 3)
                dlo = dv & (SUB - 1)
                for tv in tvs:
                    a = plsc.load_gather(r0, [tv, dv])
                    b = plsc.load_gather(r1, [tv, dv])
                    plsc.store_scatter(out_v, [dhi, dlo, tv], a + b)
                return c2

            lax.fori_loop(0, DIM, diag_body, 0, unroll=2)
            pltpu.sync_copy(out_v, out_slice(kb))
            return carry

        lax.fori_loop(0, per_w, body, 0)

    return k


def kernel(batch_pos_list, table):
    B, L, P = batch_pos_list.shape
    assert P == 2
    V, D = table.shape
    assert D == DIM
    idx2 = (batch_pos_list.transpose(1, 2, 0)
            .reshape(L, P, B // BLK, BLK)
            .transpose(0, 2, 1, 3)
            .reshape(L * P * (B // BLK), BLK))
    k = _make_kernel(B, L, V)
    out4 = k(idx2, table)
    return (out4.transpose(0, 2, 1, 3)
            .reshape(L, DIM, B)
            .transpose(2, 0, 1))


# speedup vs baseline: 1.0458x; 1.0458x over previous
import functools

import jax
import jax.numpy as jnp
from jax import lax
from jax.experimental import pallas as pl
from jax.experimental.pallas import tpu as pltpu
from jax.experimental.pallas import tpu_sc as plsc

DIM = 64
LANES = 16
BLK = 128
SUB = 8


def _make_kernel(B, L, vocab):
    info = plsc.get_sparse_core_info()
    num_workers = info.num_cores * info.num_subcores
    nB = B // BLK
    assert nB * BLK == B
    n_blocks = L * nB
    per_w = n_blocks // num_workers
    assert per_w * num_workers == n_blocks
    assert per_w % 2 == 0

    mesh = plsc.VectorSubcoreMesh(core_axis_name="c", subcore_axis_name="s")

    @functools.partial(
        pl.kernel,
        mesh=mesh,
        out_type=jax.ShapeDtypeStruct((L * DIM // SUB, nB, SUB, BLK),
                                      jnp.float32),
        scratch_types=[
            pltpu.VMEM((per_w, BLK), jnp.int32),
            pltpu.VMEM((4, BLK, DIM), jnp.float32),
            pltpu.VMEM((SUB, SUB, BLK), jnp.float32),
            pltpu.SemaphoreType.DMA((2,)),
        ],
        compiler_params=pltpu.CompilerParams(
            use_tc_tiling_on_sc=False, needs_layout_passes=False),
    )
    def k(idx_hbm, table_hbm, out_hbm, idx_v, rows, out_v, semg):
        wid = lax.axis_index("s") * info.num_cores + lax.axis_index("c")
        lane = lax.iota(jnp.int32, LANES)

        HB = per_w // 2

        def load_idx_half(h):
            pltpu.sync_copy(
                idx_hbm.at[pl.ds(
                    pl.multiple_of(wid * 2 * per_w + h * 2 * HB, 8), 2 * HB)],
                idx_v)

        load_idx_half(0)

        def fire(kb, par):
            r = lax.rem(2 * kb, 2 * HB)
            pltpu.async_copy(table_hbm.at[idx_v.at[r]],
                             rows.at[2 * par], semg.at[par])
            pltpu.async_copy(table_hbm.at[idx_v.at[r + 1]],
                             rows.at[2 * par + 1], semg.at[par])

        def wait(kb, par):
            r = lax.rem(2 * kb, 2 * HB)
            pltpu.make_async_copy(table_hbm.at[idx_v.at[r]],
                                  rows.at[2 * par], semg.at[par]).wait()
            pltpu.make_async_copy(table_hbm.at[idx_v.at[r + 1]],
                                  rows.at[2 * par + 1], semg.at[par]).wait()

        def out_slice(kb):
            gb = wid * per_w + kb
            l = gb // nB
            c = gb % nB
            return out_hbm.at[pl.ds(pl.multiple_of(l * SUB, 8), SUB), c]

        fire(0, 0)

        def body(kb, carry):
            par = lax.rem(kb, 2)
            nxt = 1 - par

            @pl.when(jnp.logical_and(kb < per_w - 1, kb != HB - 1))
            def _():
                fire(kb + 1, nxt)

            wait(kb, par)

            @pl.when(kb == HB - 1)
            def _():
                load_idx_half(1)
                fire(kb + 1, nxt)

            r0 = rows.at[2 * par]
            r1 = rows.at[2 * par + 1]
            tvs = [lane + t0 for t0 in range(0, BLK, LANES)]

            def diag_body(j, c2):
                dv = ((lane + j) & (LANES - 1)) + (j & (DIM - LANES))
                dhi = lax.shift_right_logical(dv, 3)
                dlo = dv & (SUB - 1)
                for tv in tvs:
                    a = plsc.load_gather(r0, [tv, dv])
                    b = plsc.load_gather(r1, [tv, dv])
                    plsc.store_scatter(out_v, [dhi, dlo, tv], a + b)
                return c2

            lax.fori_loop(0, DIM, diag_body, 0, unroll=4)
            pltpu.sync_copy(out_v, out_slice(kb))
            return carry

        lax.fori_loop(0, per_w, body, 0)

    return k


def kernel(batch_pos_list, table):
    B, L, P = batch_pos_list.shape
    assert P == 2
    V, D = table.shape
    assert D == DIM
    idx2 = (batch_pos_list.transpose(1, 2, 0)
            .reshape(L, P, B // BLK, BLK)
            .transpose(0, 2, 1, 3)
            .reshape(L * P * (B // BLK), BLK))
    k = _make_kernel(B, L, V)
    out4 = k(idx2, table)
    return (out4.transpose(0, 2, 1, 3)
            .reshape(L, DIM, B)
            .transpose(2, 0, 1))


# speedup vs baseline: 1.0609x; 1.0144x over previous
import functools

import jax
import jax.numpy as jnp
from jax import lax
from jax.experimental import pallas as pl
from jax.experimental.pallas import tpu as pltpu
from jax.experimental.pallas import tpu_sc as plsc

DIM = 64
LANES = 16
BLK = 128
SUB = 8


def _make_kernel(B, L, vocab):
    info = plsc.get_sparse_core_info()
    num_workers = info.num_cores * info.num_subcores
    nB = B // BLK
    assert nB * BLK == B
    n_blocks = L * nB
    per_w = n_blocks // num_workers
    assert per_w * num_workers == n_blocks
    assert per_w % 2 == 0

    mesh = plsc.VectorSubcoreMesh(core_axis_name="c", subcore_axis_name="s")

    @functools.partial(
        pl.kernel,
        mesh=mesh,
        out_type=jax.ShapeDtypeStruct((L * DIM // SUB, nB, SUB, BLK),
                                      jnp.float32),
        scratch_types=[
            pltpu.VMEM((per_w, BLK), jnp.int32),
            pltpu.VMEM((4, BLK, DIM), jnp.float32),
            pltpu.VMEM((SUB, SUB, BLK), jnp.float32),
            pltpu.SemaphoreType.DMA((2,)),
        ],
        compiler_params=pltpu.CompilerParams(
            use_tc_tiling_on_sc=False, needs_layout_passes=False),
    )
    def k(idx_hbm, table_hbm, out_hbm, idx_v, rows, out_v, semg):
        wid = lax.axis_index("s") * info.num_cores + lax.axis_index("c")
        lane = lax.iota(jnp.int32, LANES)

        HB = per_w // 2

        def load_idx_half(h):
            pltpu.sync_copy(
                idx_hbm.at[pl.ds(
                    pl.multiple_of(wid * 2 * per_w + h * 2 * HB, 8), 2 * HB)],
                idx_v)

        load_idx_half(0)

        def fire(kb, par):
            r = lax.rem(2 * kb, 2 * HB)
            pltpu.async_copy(table_hbm.at[idx_v.at[r]],
                             rows.at[2 * par], semg.at[par])
            pltpu.async_copy(table_hbm.at[idx_v.at[r + 1]],
                             rows.at[2 * par + 1], semg.at[par])

        def wait(kb, par):
            r = lax.rem(2 * kb, 2 * HB)
            pltpu.make_async_copy(table_hbm.at[idx_v.at[r]],
                                  rows.at[2 * par], semg.at[par]).wait()
            pltpu.make_async_copy(table_hbm.at[idx_v.at[r + 1]],
                                  rows.at[2 * par + 1], semg.at[par]).wait()

        def out_slice(kb):
            gb = wid * per_w + kb
            l = gb // nB
            c = gb % nB
            return out_hbm.at[pl.ds(pl.multiple_of(l * SUB, 8), SUB), c]

        fire(0, 0)

        def body(kb, carry):
            par = lax.rem(kb, 2)
            nxt = 1 - par

            @pl.when(jnp.logical_and(kb < per_w - 1, kb != HB - 1))
            def _():
                fire(kb + 1, nxt)

            wait(kb, par)

            @pl.when(kb == HB - 1)
            def _():
                load_idx_half(1)
                fire(kb + 1, nxt)

            r0 = rows.at[2 * par]
            r1 = rows.at[2 * par + 1]
            tvs = [lane + t0 for t0 in range(0, BLK, LANES)]

            def diag_body(j, c2):
                dv = ((lane + j) & (LANES - 1)) + (j & (DIM - LANES))
                dhi = lax.shift_right_logical(dv, 3)
                dlo = dv & (SUB - 1)
                for tv in tvs:
                    a = plsc.load_gather(r0, [tv, dv])
                    b = plsc.load_gather(r1, [tv, dv])
                    plsc.store_scatter(out_v, [dhi, dlo, tv], a + b)
                return c2

            lax.fori_loop(0, DIM, diag_body, 0, unroll=8)
            pltpu.sync_copy(out_v, out_slice(kb))
            return carry

        lax.fori_loop(0, per_w, body, 0)

    return k


def kernel(batch_pos_list, table):
    B, L, P = batch_pos_list.shape
    assert P == 2
    V, D = table.shape
    assert D == DIM
    idx2 = (batch_pos_list.transpose(1, 2, 0)
            .reshape(L, P, B // BLK, BLK)
            .transpose(0, 2, 1, 3)
            .reshape(L * P * (B // BLK), BLK))
    k = _make_kernel(B, L, V)
    out4 = k(idx2, table)
    return (out4.transpose(0, 2, 1, 3)
            .reshape(L, DIM, B)
            .transpose(2, 0, 1))


# speedup vs baseline: 1.1822x; 1.1143x over previous
import functools

import jax
import jax.numpy as jnp
from jax import lax
from jax.experimental import pallas as pl
from jax.experimental.pallas import tpu as pltpu
from jax.experimental.pallas import tpu_sc as plsc

DIM = 64
LANES = 16
BLK = 128
SUB = 8


def _make_kernel(B, L, vocab):
    info = plsc.get_sparse_core_info()
    num_workers = info.num_cores * info.num_subcores
    nB = B // BLK
    assert nB * BLK == B
    n_blocks = L * nB
    per_w = n_blocks // num_workers
    assert per_w * num_workers == n_blocks
    assert per_w % 2 == 0

    mesh = plsc.VectorSubcoreMesh(core_axis_name="c", subcore_axis_name="s")

    @functools.partial(
        pl.kernel,
        mesh=mesh,
        out_type=jax.ShapeDtypeStruct((L * DIM // SUB, nB, SUB, BLK),
                                      jnp.float32),
        scratch_types=[
            pltpu.VMEM((per_w, BLK), jnp.int32),
            pltpu.VMEM((4, BLK, DIM), jnp.float32),
            pltpu.VMEM((2, SUB, SUB, BLK), jnp.float32),
            pltpu.SemaphoreType.DMA((2,)),
            pltpu.SemaphoreType.DMA((2,)),
        ],
        compiler_params=pltpu.CompilerParams(
            use_tc_tiling_on_sc=False, needs_layout_passes=False),
    )
    def k(idx_hbm, table_hbm, out_hbm, idx_v, rows, outb, semg, semo):
        wid = lax.axis_index("s") * info.num_cores + lax.axis_index("c")
        lane = lax.iota(jnp.int32, LANES)

        HB = per_w // 2

        def load_idx_half(h):
            pltpu.sync_copy(
                idx_hbm.at[pl.ds(
                    pl.multiple_of(wid * 2 * per_w + h * 2 * HB, 8), 2 * HB)],
                idx_v)

        load_idx_half(0)

        def fire(kb, par):
            r = lax.rem(2 * kb, 2 * HB)
            pltpu.async_copy(table_hbm.at[idx_v.at[r]],
                             rows.at[2 * par], semg.at[par])
            pltpu.async_copy(table_hbm.at[idx_v.at[r + 1]],
                             rows.at[2 * par + 1], semg.at[par])

        def wait(kb, par):
            r = lax.rem(2 * kb, 2 * HB)
            pltpu.make_async_copy(table_hbm.at[idx_v.at[r]],
                                  rows.at[2 * par], semg.at[par]).wait()
            pltpu.make_async_copy(table_hbm.at[idx_v.at[r + 1]],
                                  rows.at[2 * par + 1], semg.at[par]).wait()

        def out_slice(kb):
            gb = wid * per_w + kb
            l = gb // nB
            c = gb % nB
            return out_hbm.at[pl.ds(pl.multiple_of(l * SUB, 8), SUB), c]

        fire(0, 0)

        def body(kb, carry):
            par = lax.rem(kb, 2)
            nxt = 1 - par

            @pl.when(jnp.logical_and(kb < per_w - 1, kb != HB - 1))
            def _():
                fire(kb + 1, nxt)

            wait(kb, par)

            @pl.when(kb == HB - 1)
            def _():
                load_idx_half(1)
                fire(kb + 1, nxt)

            @pl.when(kb >= 2)
            def _():
                pltpu.make_async_copy(outb.at[par], out_slice(kb - 2),
                                      semo.at[par]).wait()

            r0 = rows.at[2 * par]
            r1 = rows.at[2 * par + 1]
            out_v = outb.at[par]
            tvs = [lane + t0 for t0 in range(0, BLK, LANES)]

            def diag_body(j, c2):
                dv = ((lane + j) & (LANES - 1)) + (j & (DIM - LANES))
                dhi = lax.shift_right_logical(dv, 3)
                dlo = dv & (SUB - 1)
                for tv in tvs:
                    a = plsc.load_gather(r0, [tv, dv])
                    b = plsc.load_gather(r1, [tv, dv])
                    plsc.store_scatter(out_v, [dhi, dlo, tv], a + b)
                return c2

            lax.fori_loop(0, DIM, diag_body, 0, unroll=8)
            pltpu.async_copy(out_v, out_slice(kb), semo.at[par])
            return carry

        lax.fori_loop(0, per_w, body, 0)
        for kb in (per_w - 2, per_w - 1):
            pltpu.make_async_copy(outb.at[kb % 2], out_slice(kb),
                                  semo.at[kb % 2]).wait()

    return k


def kernel(batch_pos_list, table):
    B, L, P = batch_pos_list.shape
    assert P == 2
    V, D = table.shape
    assert D == DIM
    idx2 = (batch_pos_list.transpose(1, 2, 0)
            .reshape(L, P, B // BLK, BLK)
            .transpose(0, 2, 1, 3)
            .reshape(L * P * (B // BLK), BLK))
    k = _make_kernel(B, L, V)
    out4 = k(idx2, table)
    return (out4.transpose(0, 2, 1, 3)
            .reshape(L, DIM, B)
            .transpose(2, 0, 1))
